# Initial kernel scaffold; baseline (speedup 1.0000x reference)
import jax
import jax.numpy as jnp
from jax.experimental import pallas as pl


def kernel(x, candidate_x, W_embed, b_embed, bn1_g, bn1_b, W_mb1, b_mb1, W_mb2, b_mb2, bn2_g, bn2_b, W_mlp0, b_mlp0, W_mlp1, b_mlp1, W_out, b_out, is_train):
    raise NotImplementedError("write your pallas kernel here")


# speedup vs baseline: 6.0505x; 6.0505x over previous
import functools

import jax
import jax.numpy as jnp
from jax import lax
from jax.experimental import pallas as pl
from jax.experimental.pallas import tpu as pltpu
from jax.experimental.pallas import tpu_sc as plsc

B = 1024
N = 100000
D = 128
K = 32
EPS = 1e-5

NPAD = 102400
CB = 4096
QB = 256
SG = 128
NSG = NPAD // SG
SPB = CB // SG
GMW = (NPAD // CB) * SG
POOL = K * SG

_F32 = jnp.float32
_I32 = jnp.int32


def _dott(a, b):
    return lax.dot_general(a, b, (((1,), (1,)), ((), ())),
                           preferred_element_type=_F32)


def _embed_block(h, we, be, g1, b1, wm1, bm1, wm2, bm2, g2, b2):
    inv = 1.0 / jnp.sqrt(jnp.asarray(1.0 + EPS, _F32))
    h = _dott(h, we) + be
    z = g1 * (h * inv) + b1
    z = jnp.maximum(_dott(z, wm1) + bm1, 0.0)
    h = _dott(z, wm2) + bm2
    return g2 * (h * inv) + b2


def _xside_body(x_ref, we, be, g1, b1, wm1, bm1, wm2, bm2, g2, b2, w0a, b0,
                xe_ref, qn_ref, a0_ref):
    h = _embed_block(x_ref[...], we[...], be[...], g1[...], b1[...], wm1[...],
                     bm1[...], wm2[...], bm2[...], g2[...], b2[...])
    xe_ref[...] = h
    qn_ref[...] = jnp.sum(h * h, axis=1, keepdims=True)
    a0_ref[...] = _dott(h, w0a[...]) + b0[...]


def _xside(x, we, be, g1, b1, wm1, bm1, wm2, bm2, g2, b2, w0a, b0):
    return pl.pallas_call(
        _xside_body,
        out_shape=(
            jax.ShapeDtypeStruct((B, D), _F32),
            jax.ShapeDtypeStruct((B, 1), _F32),
            jax.ShapeDtypeStruct((B, 2 * D), _F32),
        ),
    )(x, we, be, g1, b1, wm1, bm1, wm2, bm2, g2, b2, w0a, b0)


def _cemb_body(cx_ref, we, be, g1, b1, wm1, bm1, wm2, bm2, g2, b2,
               ce_ref, cn_ref):
    h = _embed_block(cx_ref[...], we[...], be[...], g1[...], b1[...],
                     wm1[...], bm1[...], wm2[...], bm2[...], g2[...], b2[...])
    ce_ref[...] = h
    cn = jnp.sum(h * h, axis=1, keepdims=True)
    grow = (lax.broadcasted_iota(_I32, (1024, 1), 0)
            + pl.program_id(0) * 1024)
    cn_ref[...] = jnp.where(grow >= N, 1e30, cn)


def _cand_embed(cx_pad, we, be, g1, b1, wm1, bm1, wm2, bm2, g2, b2):
    nblk = NPAD // 1024
    w = lambda s: pl.BlockSpec(s, lambda i: (0,) * len(s))
    return pl.pallas_call(
        _cemb_body,
        grid=(nblk,),
        in_specs=[pl.BlockSpec((1024, D), lambda i: (i, 0)),
                  w((D, D)), w((1, D)), w((1, D)), w((1, D)), w((D, D)),
                  w((1, D)), w((D, D)), w((1, D)), w((1, D)), w((1, D))],
        out_specs=(pl.BlockSpec((1024, D), lambda i: (i, 0)),
                   pl.BlockSpec((1024, 1), lambda i: (i, 0))),
        out_shape=(jax.ShapeDtypeStruct((NPAD, D), _F32),
                   jax.ShapeDtypeStruct((NPAD, 1), _F32)),
        compiler_params=pltpu.CompilerParams(
            dimension_semantics=("parallel",)),
    )(cx_pad, we, be, g1, b1, wm1, bm1, wm2, bm2, g2, b2)


def _dist_body(xe_ref, qn_ref, ce_ref, cn_ref, d2_ref, gm_ref):
    x = xe_ref[...]
    c = ce_ref[...]
    s = _dott(x, c)
    d2 = (qn_ref[...] - 2.0 * s) + cn_ref[...]
    d2_ref[...] = d2
    gsm = jnp.concatenate(
        [jnp.min(d2[:, a * SG:(a + 1) * SG], axis=1, keepdims=True)
         for a in range(SPB)], axis=1)
    gm_ref[...] = jnp.concatenate(
        [gsm, jnp.full((QB, SG - SPB), jnp.inf, _F32)], axis=1)


def _dist(xe, qn, ce, cnr):
    return pl.pallas_call(
        _dist_body,
        grid=(B // QB, NPAD // CB),
        in_specs=[pl.BlockSpec((QB, D), lambda i, j: (i, 0)),
                  pl.BlockSpec((QB, 1), lambda i, j: (i, 0)),
                  pl.BlockSpec((CB, D), lambda i, j: (j, 0)),
                  pl.BlockSpec((1, CB), lambda i, j: (0, j))],
        out_specs=(pl.BlockSpec((QB, CB), lambda i, j: (i, j)),
                   pl.BlockSpec((QB, SG), lambda i, j: (i, j))),
        out_shape=(jax.ShapeDtypeStruct((B, NPAD), _F32),
                   jax.ShapeDtypeStruct((B, GMW), _F32)),
        compiler_params=pltpu.CompilerParams(
            dimension_semantics=("parallel", "parallel")),
    )(xe, qn, ce, cnr)


def _gtopk_body(gm_ref, gid_ref, pg_ref):
    v0 = gm_ref[...]
    lane = lax.broadcasted_iota(_I32, (QB, GMW), 1)
    col = lax.broadcasted_iota(_I32, (QB, K), 1)
    qrow = (lax.broadcasted_iota(_I32, (QB, 1), 0)
            + pl.program_id(0) * QB)
    big = jnp.int32(2 ** 30)
    zer = jnp.zeros((QB, K), _I32)

    def step(t, carry):
        v, gidm, pgm = carry
        m = jnp.min(v, axis=1, keepdims=True)
        am = jnp.min(jnp.where(v <= m, lane, big), axis=1, keepdims=True)
        sr = (am // SG) * SPB + am % SG
        sel = col == t
        gidm = jnp.where(sel, sr, gidm)
        pgm = jnp.where(sel, sr + qrow * NSG, pgm)
        v = jnp.where(lane == am, jnp.inf, v)
        return v, gidm, pgm

    _, gidm, pgm = lax.fori_loop(0, K, step, (v0, zer, zer))
    gid_ref[...] = gidm
    pg_ref[...] = pgm


def _gtopk(gm):
    return pl.pallas_call(
        _gtopk_body,
        grid=(B // QB,),
        in_specs=[pl.BlockSpec((QB, GMW), lambda i: (i, 0))],
        out_specs=(pl.BlockSpec((QB, K), lambda i: (i, 0)),
                   pl.BlockSpec((QB, K), lambda i: (i, 0))),
        out_shape=(jax.ShapeDtypeStruct((B, K), _I32),
                   jax.ShapeDtypeStruct((B, K), _I32)),
        compiler_params=pltpu.CompilerParams(
            dimension_semantics=("parallel",)),
    )(gm)


_NW = 32


def _sc_mesh():
    return plsc.VectorSubcoreMesh(core_axis_name="c", subcore_axis_name="s",
                                  num_cores=2, num_subcores=16)


def _sc_row_gather(tab, idx_flat, nrows_tab):
    nrows = B * K
    bpw = nrows // _NW
    npass = 4
    rpp = bpw // npass

    @functools.partial(
        pl.kernel, mesh=_sc_mesh(),
        out_type=jax.ShapeDtypeStruct((nrows, SG), _F32),
        scratch_types=[pltpu.VMEM((bpw,), _I32),
                       pltpu.VMEM((rpp, SG), _F32),
                       pltpu.SemaphoreType.DMA],
    )
    def k(tab_hbm, idx_hbm, out_hbm, idx_v, rows_v, sem):
        wid = lax.axis_index("s") * 2 + lax.axis_index("c")
        base = wid * bpw
        pltpu.sync_copy(idx_hbm.at[pl.ds(base, bpw)], idx_v)
        for p in range(npass):
            cps = [pltpu.async_copy(
                       tab_hbm.at[idx_v.at[pl.ds(p * rpp + ci * 128, 128)]],
                       rows_v.at[pl.ds(ci * 128, 128)], sem)
                   for ci in range(rpp // 128)]
            for cp in cps:
                cp.wait()
            pltpu.sync_copy(rows_v, out_hbm.at[pl.ds(base + p * rpp, rpp)])

    return k(tab, idx_flat)


def _pool_gather(d2rows, pgidx_flat):
    return _sc_row_gather(d2rows, pgidx_flat, B * NSG)


def _ptopk_body(pool_ref, gid_ref, idx_ref):
    v0 = pool_ref[...]
    g = gid_ref[...].astype(_F32)
    jj = lax.broadcasted_iota(_I32, (K, POOL), 0)
    ll = lax.broadcasted_iota(_I32, (K, POOL), 1)
    eye = jnp.where(ll // SG == jj, 1.0, 0.0).astype(_F32)
    gb = lax.dot_general(g, eye, (((1,), (0,)), ((), ())),
                         precision=lax.Precision.HIGHEST,
                         preferred_element_type=_F32)
    idxmat = ((gb + 0.5).astype(_I32) * SG
              + lax.broadcasted_iota(_I32, (QB, POOL), 1) % SG)
    big = jnp.int32(2 ** 30)
    col = lax.broadcasted_iota(_I32, (QB, K), 1)

    def step(t, carry):
        v, idxm = carry
        m = jnp.min(v, axis=1, keepdims=True)
        am = jnp.min(jnp.where(v <= m, idxmat, big), axis=1, keepdims=True)
        idxm = jnp.where(col == t, am, idxm)
        v = jnp.where(idxmat == am, jnp.inf, v)
        return v, idxm

    _, idxm = lax.fori_loop(0, K, step, (v0, jnp.zeros((QB, K), _I32)))
    idx_ref[...] = idxm


def _ptopk(pool, gids):
    return pl.pallas_call(
        _ptopk_body,
        grid=(B // QB,),
        in_specs=[pl.BlockSpec((QB, POOL), lambda i: (i, 0)),
                  pl.BlockSpec((QB, K), lambda i: (i, 0))],
        out_specs=pl.BlockSpec((QB, K), lambda i: (i, 0)),
        out_shape=jax.ShapeDtypeStruct((B, K), _I32),
        compiler_params=pltpu.CompilerParams(
            dimension_semantics=("parallel",)),
    )(pool, gids)


def _ctx_gather(cand_emb, idx_flat):
    return _sc_row_gather(cand_emb, idx_flat, NPAD)


def _mlp_body(ctx_ref, a0_ref, w0b, w1, b1, wo, bo, out_ref):
    ctx = ctx_ref[...]
    a0 = a0_ref[...]
    rr = lax.broadcasted_iota(_I32, (ctx.shape[0], a0.shape[0]), 0) // K
    jj = lax.broadcasted_iota(_I32, (ctx.shape[0], a0.shape[0]), 1)
    rep = jnp.where(rr == jj, 1.0, 0.0).astype(_F32)
    aexp = lax.dot_general(rep, a0, (((1,), (0,)), ((), ())),
                           precision=lax.Precision.HIGHEST,
                           preferred_element_type=_F32)
    h1 = jnp.maximum(aexp + _dott(ctx, w0b[...]), 0.0)
    h2 = jnp.maximum(_dott(h1, w1[...]) + b1[...], 0.0)
    out_ref[...] = jnp.sum(h2 * wo[...], axis=1, keepdims=True) + bo[0]


def _mlp(ctx, a0, w0b, w1, b1, wo, bo):
    RB = 512
    w = lambda s: pl.BlockSpec(s, lambda i: (0,) * len(s))
    return pl.pallas_call(
        _mlp_body,
        grid=(B * K // RB,),
        in_specs=[pl.BlockSpec((RB, D), lambda i: (i, 0)),
                  pl.BlockSpec((RB // K, 2 * D), lambda i: (i, 0)),
                  w((2 * D, D)), w((2 * D, 2 * D)), w((1, 2 * D)),
                  w((1, 2 * D)),
                  pl.BlockSpec(memory_space=pltpu.SMEM)],
        out_specs=pl.BlockSpec((RB, 1), lambda i: (i, 0)),
        out_shape=jax.ShapeDtypeStruct((B * K, 1), _F32),
        compiler_params=pltpu.CompilerParams(
            dimension_semantics=("parallel",)),
    )(ctx, a0, w0b, w1, b1, wo, bo)


def kernel(x, candidate_x, W_embed, b_embed, bn1_g, bn1_b, W_mb1, b_mb1,
           W_mb2, b_mb2, bn2_g, bn2_b, W_mlp0, b_mlp0, W_mlp1, b_mlp1,
           W_out, b_out, is_train):
    del is_train
    row = lambda v: v.reshape(1, -1)
    be, g1, b1 = row(b_embed), row(bn1_g), row(bn1_b)
    bm1, bm2 = row(b_mb1), row(b_mb2)
    g2, b2 = row(bn2_g), row(bn2_b)
    w0a, w0b = W_mlp0[:, :D], W_mlp0[:, D:]
    b0, b1m, bo = row(b_mlp0), row(b_mlp1), b_out

    xe, qn, a0 = _xside(x, W_embed, be, g1, b1, W_mb1, bm1, W_mb2, bm2,
                        g2, b2, w0a, b0)
    cx_pad = jnp.pad(candidate_x, ((0, NPAD - N), (0, 0)))
    ce, cn = _cand_embed(cx_pad, W_embed, be, g1, b1, W_mb1, bm1, W_mb2, bm2,
                         g2, b2)
    d2, gm = _dist(xe, qn, ce, cn.reshape(1, NPAD))
    gids, pgidx = _gtopk(gm)
    pool = _pool_gather(d2.reshape(B * NSG, SG), pgidx.reshape(-1))
    idx = _ptopk(pool.reshape(B, POOL), gids)
    ctx = _ctx_gather(ce, idx.reshape(-1))
    out = _mlp(ctx, a0, w0b, W_mlp1, b1m, W_out, bo)
    return out.reshape(B, K, 1)


# speedup vs baseline: 6.7934x; 1.1228x over previous
import functools

import jax
import jax.numpy as jnp
from jax import lax
from jax.experimental import pallas as pl
from jax.experimental.pallas import tpu as pltpu
from jax.experimental.pallas import tpu_sc as plsc

B = 1024
N = 100000
D = 128
K = 32
EPS = 1e-5

NPAD = 102400
CB = 4096
QB = 256
SG = 128
NSG = NPAD // SG
SPB = CB // SG
GMW = (NPAD // CB) * SG
POOL = K * SG

_F32 = jnp.float32
_I32 = jnp.int32


def _dott(a, b):
    return lax.dot_general(a, b, (((1,), (1,)), ((), ())),
                           preferred_element_type=_F32)


def _embed_block(h, we, be, g1, b1, wm1, bm1, wm2, bm2, g2, b2):
    inv = 1.0 / jnp.sqrt(jnp.asarray(1.0 + EPS, _F32))
    h = _dott(h, we) + be
    z = g1 * (h * inv) + b1
    z = jnp.maximum(_dott(z, wm1) + bm1, 0.0)
    h = _dott(z, wm2) + bm2
    return g2 * (h * inv) + b2


def _xside_body(x_ref, we, be, g1, b1, wm1, bm1, wm2, bm2, g2, b2, w0a, b0,
                xe_ref, qn_ref, a0_ref):
    h = _embed_block(x_ref[...], we[...], be[...], g1[...], b1[...], wm1[...],
                     bm1[...], wm2[...], bm2[...], g2[...], b2[...])
    xe_ref[...] = h
    qn_ref[...] = jnp.sum(h * h, axis=1, keepdims=True)
    a0_ref[...] = _dott(h, w0a[...]) + b0[...]


def _xside(x, we, be, g1, b1, wm1, bm1, wm2, bm2, g2, b2, w0a, b0):
    return pl.pallas_call(
        _xside_body,
        out_shape=(
            jax.ShapeDtypeStruct((B, D), _F32),
            jax.ShapeDtypeStruct((B, 1), _F32),
            jax.ShapeDtypeStruct((B, 2 * D), _F32),
        ),
    )(x, we, be, g1, b1, wm1, bm1, wm2, bm2, g2, b2, w0a, b0)


def _cemb_body(cx_ref, we, be, g1, b1, wm1, bm1, wm2, bm2, g2, b2,
               ce_ref, cn_ref):
    h = _embed_block(cx_ref[...], we[...], be[...], g1[...], b1[...],
                     wm1[...], bm1[...], wm2[...], bm2[...], g2[...], b2[...])
    ce_ref[...] = h
    cn = jnp.sum(h * h, axis=1, keepdims=True)
    grow = (lax.broadcasted_iota(_I32, (1024, 1), 0)
            + pl.program_id(0) * 1024)
    cn_ref[...] = jnp.where(grow >= N, 1e30, cn)


def _cand_embed(cx_pad, we, be, g1, b1, wm1, bm1, wm2, bm2, g2, b2):
    nblk = NPAD // 1024
    w = lambda s: pl.BlockSpec(s, lambda i: (0,) * len(s))
    return pl.pallas_call(
        _cemb_body,
        grid=(nblk,),
        in_specs=[pl.BlockSpec((1024, D), lambda i: (i, 0)),
                  w((D, D)), w((1, D)), w((1, D)), w((1, D)), w((D, D)),
                  w((1, D)), w((D, D)), w((1, D)), w((1, D)), w((1, D))],
        out_specs=(pl.BlockSpec((1024, D), lambda i: (i, 0)),
                   pl.BlockSpec((1024, 1), lambda i: (i, 0))),
        out_shape=(jax.ShapeDtypeStruct((NPAD, D), _F32),
                   jax.ShapeDtypeStruct((NPAD, 1), _F32)),
        compiler_params=pltpu.CompilerParams(
            dimension_semantics=("parallel",)),
    )(cx_pad, we, be, g1, b1, wm1, bm1, wm2, bm2, g2, b2)


def _dist_body(xe_ref, qn_ref, ce_ref, cn_ref, d2_ref, gm_ref):
    x = xe_ref[...]
    c = ce_ref[...]
    s = _dott(x, c)
    d2 = (qn_ref[...] - 2.0 * s) + cn_ref[...]
    d2_ref[...] = d2
    gsm = jnp.concatenate(
        [jnp.min(d2[:, a * SG:(a + 1) * SG], axis=1, keepdims=True)
         for a in range(SPB)], axis=1)
    gm_ref[...] = jnp.concatenate(
        [gsm, jnp.full((QB, SG - SPB), jnp.inf, _F32)], axis=1)


def _dist(xe, qn, ce, cnr):
    return pl.pallas_call(
        _dist_body,
        grid=(B // QB, NPAD // CB),
        in_specs=[pl.BlockSpec((QB, D), lambda i, j: (i, 0)),
                  pl.BlockSpec((QB, 1), lambda i, j: (i, 0)),
                  pl.BlockSpec((CB, D), lambda i, j: (j, 0)),
                  pl.BlockSpec((1, CB), lambda i, j: (0, j))],
        out_specs=(pl.BlockSpec((QB, CB), lambda i, j: (i, j)),
                   pl.BlockSpec((QB, SG), lambda i, j: (i, j))),
        out_shape=(jax.ShapeDtypeStruct((B, NPAD), _F32),
                   jax.ShapeDtypeStruct((B, GMW), _F32)),
        compiler_params=pltpu.CompilerParams(
            dimension_semantics=("parallel", "parallel")),
    )(xe, qn, ce, cnr)


def _gtopk_body(gm_ref, gid_ref, pg_ref):
    v0 = gm_ref[...]
    lane = lax.broadcasted_iota(_I32, (QB, GMW), 1)
    col = lax.broadcasted_iota(_I32, (QB, K), 1)
    qrow = (lax.broadcasted_iota(_I32, (QB, 1), 0)
            + pl.program_id(0) * QB)
    big = jnp.int32(2 ** 30)
    zer = jnp.zeros((QB, K), _I32)

    def step(t, carry):
        v, gidm, pgm = carry
        m = jnp.min(v, axis=1, keepdims=True)
        am = jnp.min(jnp.where(v <= m, lane, big), axis=1, keepdims=True)
        sr = (am // SG) * SPB + am % SG
        sel = col == t
        gidm = jnp.where(sel, sr, gidm)
        pgm = jnp.where(sel, sr + qrow * NSG, pgm)
        v = jnp.where(lane == am, jnp.inf, v)
        return v, gidm, pgm

    _, gidm, pgm = lax.fori_loop(0, K, step, (v0, zer, zer), unroll=4)
    gid_ref[...] = gidm
    pg_ref[...] = pgm


def _gtopk(gm):
    return pl.pallas_call(
        _gtopk_body,
        grid=(B // QB,),
        in_specs=[pl.BlockSpec((QB, GMW), lambda i: (i, 0))],
        out_specs=(pl.BlockSpec((QB, K), lambda i: (i, 0)),
                   pl.BlockSpec((QB, K), lambda i: (i, 0))),
        out_shape=(jax.ShapeDtypeStruct((B, K), _I32),
                   jax.ShapeDtypeStruct((B, K), _I32)),
        compiler_params=pltpu.CompilerParams(
            dimension_semantics=("parallel",)),
    )(gm)


_NW = 32


def _sc_mesh():
    return plsc.VectorSubcoreMesh(core_axis_name="c", subcore_axis_name="s",
                                  num_cores=2, num_subcores=16)


def _sc_row_gather(tab, idx_flat, nrows_tab):
    nrows = B * K
    bpw = nrows // _NW
    npass = 4
    rpp = bpw // npass

    @functools.partial(
        pl.kernel, mesh=_sc_mesh(),
        out_type=jax.ShapeDtypeStruct((nrows, SG), _F32),
        scratch_types=[pltpu.VMEM((bpw,), _I32),
                       pltpu.VMEM((rpp, SG), _F32),
                       pltpu.SemaphoreType.DMA],
    )
    def k(tab_hbm, idx_hbm, out_hbm, idx_v, rows_v, sem):
        wid = lax.axis_index("s") * 2 + lax.axis_index("c")
        base = wid * bpw
        pltpu.sync_copy(idx_hbm.at[pl.ds(base, bpw)], idx_v)
        for p in range(npass):
            cps = [pltpu.async_copy(
                       tab_hbm.at[idx_v.at[pl.ds(p * rpp + ci * 128, 128)]],
                       rows_v.at[pl.ds(ci * 128, 128)], sem)
                   for ci in range(rpp // 128)]
            for cp in cps:
                cp.wait()
            pltpu.sync_copy(rows_v, out_hbm.at[pl.ds(base + p * rpp, rpp)])

    return k(tab, idx_flat)


def _pool_gather(d2rows, pgidx_flat):
    return _sc_row_gather(d2rows, pgidx_flat, B * NSG)


def _ptopk_body(pool_ref, gid_ref, idx_ref):
    v0 = pool_ref[...]
    g = gid_ref[...].astype(_F32)
    jj = lax.broadcasted_iota(_I32, (K, POOL), 0)
    ll = lax.broadcasted_iota(_I32, (K, POOL), 1)
    eye = jnp.where(ll // SG == jj, 1.0, 0.0).astype(_F32)
    gb = lax.dot_general(g, eye, (((1,), (0,)), ((), ())),
                         precision=lax.Precision.HIGHEST,
                         preferred_element_type=_F32)
    idxmat = ((gb + 0.5).astype(_I32) * SG
              + lax.broadcasted_iota(_I32, (QB, POOL), 1) % SG)
    big = jnp.int32(2 ** 30)
    col = lax.broadcasted_iota(_I32, (QB, K), 1)

    def step(t, carry):
        v, idxm = carry
        m = jnp.min(v, axis=1, keepdims=True)
        am = jnp.min(jnp.where(v <= m, idxmat, big), axis=1, keepdims=True)
        idxm = jnp.where(col == t, am, idxm)
        v = jnp.where(idxmat == am, jnp.inf, v)
        return v, idxm

    _, idxm = lax.fori_loop(0, K, step, (v0, jnp.zeros((QB, K), _I32)),
                            unroll=4)
    idx_ref[...] = idxm


def _ptopk(pool, gids):
    return pl.pallas_call(
        _ptopk_body,
        grid=(B // QB,),
        in_specs=[pl.BlockSpec((QB, POOL), lambda i: (i, 0)),
                  pl.BlockSpec((QB, K), lambda i: (i, 0))],
        out_specs=pl.BlockSpec((QB, K), lambda i: (i, 0)),
        out_shape=jax.ShapeDtypeStruct((B, K), _I32),
        compiler_params=pltpu.CompilerParams(
            dimension_semantics=("parallel",)),
    )(pool, gids)


def _ctx_gather(cand_emb, idx_flat):
    return _sc_row_gather(cand_emb, idx_flat, NPAD)


def _mlp_body(ctx_ref, a0_ref, w0b, w1, b1, wo, bo, out_ref):
    ctx = ctx_ref[...]
    a0 = a0_ref[...]
    rr = lax.broadcasted_iota(_I32, (ctx.shape[0], a0.shape[0]), 0) // K
    jj = lax.broadcasted_iota(_I32, (ctx.shape[0], a0.shape[0]), 1)
    rep = jnp.where(rr == jj, 1.0, 0.0).astype(_F32)
    aexp = lax.dot_general(rep, a0, (((1,), (0,)), ((), ())),
                           precision=lax.Precision.HIGHEST,
                           preferred_element_type=_F32)
    h1 = jnp.maximum(aexp + _dott(ctx, w0b[...]), 0.0)
    h2 = jnp.maximum(_dott(h1, w1[...]) + b1[...], 0.0)
    out_ref[...] = jnp.sum(h2 * wo[...], axis=1, keepdims=True) + bo[0]


def _mlp(ctx, a0, w0b, w1, b1, wo, bo):
    RB = 512
    w = lambda s: pl.BlockSpec(s, lambda i: (0,) * len(s))
    return pl.pallas_call(
        _mlp_body,
        grid=(B * K // RB,),
        in_specs=[pl.BlockSpec((RB, D), lambda i: (i, 0)),
                  pl.BlockSpec((RB // K, 2 * D), lambda i: (i, 0)),
                  w((2 * D, D)), w((2 * D, 2 * D)), w((1, 2 * D)),
                  w((1, 2 * D)),
                  pl.BlockSpec(memory_space=pltpu.SMEM)],
        out_specs=pl.BlockSpec((RB, 1), lambda i: (i, 0)),
        out_shape=jax.ShapeDtypeStruct((B * K, 1), _F32),
        compiler_params=pltpu.CompilerParams(
            dimension_semantics=("parallel",)),
    )(ctx, a0, w0b, w1, b1, wo, bo)


def kernel(x, candidate_x, W_embed, b_embed, bn1_g, bn1_b, W_mb1, b_mb1,
           W_mb2, b_mb2, bn2_g, bn2_b, W_mlp0, b_mlp0, W_mlp1, b_mlp1,
           W_out, b_out, is_train):
    del is_train
    row = lambda v: v.reshape(1, -1)
    be, g1, b1 = row(b_embed), row(bn1_g), row(bn1_b)
    bm1, bm2 = row(b_mb1), row(b_mb2)
    g2, b2 = row(bn2_g), row(bn2_b)
    w0a, w0b = W_mlp0[:, :D], W_mlp0[:, D:]
    b0, b1m, bo = row(b_mlp0), row(b_mlp1), b_out

    xe, qn, a0 = _xside(x, W_embed, be, g1, b1, W_mb1, bm1, W_mb2, bm2,
                        g2, b2, w0a, b0)
    cx_pad = jnp.pad(candidate_x, ((0, NPAD - N), (0, 0)))
    ce, cn = _cand_embed(cx_pad, W_embed, be, g1, b1, W_mb1, bm1, W_mb2, bm2,
                         g2, b2)
    d2, gm = _dist(xe, qn, ce, cn.reshape(1, NPAD))
    gids, pgidx = _gtopk(gm)
    pool = _pool_gather(d2.reshape(B * NSG, SG), pgidx.reshape(-1))
    idx = _ptopk(pool.reshape(B, POOL), gids)
    ctx = _ctx_gather(ce, idx.reshape(-1))
    out = _mlp(ctx, a0, w0b, W_mlp1, b1m, W_out, bo)
    return out.reshape(B, K, 1)


# speedup vs baseline: 6.9279x; 1.0198x over previous
import functools

import jax
import jax.numpy as jnp
from jax import lax
from jax.experimental import pallas as pl
from jax.experimental.pallas import tpu as pltpu
from jax.experimental.pallas import tpu_sc as plsc

B = 1024
N = 100000
D = 128
K = 32
EPS = 1e-5

NPAD = 102400
CB = 4096
QB = 256
SG = 128
NSG = NPAD // SG
SPB = CB // SG
GMW = (NPAD // CB) * SG
POOL = K * SG

_F32 = jnp.float32
_I32 = jnp.int32


def _dott(a, b):
    return lax.dot_general(a, b, (((1,), (1,)), ((), ())),
                           preferred_element_type=_F32)


def _embed_block(h, we, be, g1, b1, wm1, bm1, wm2, bm2, g2, b2):
    inv = 1.0 / jnp.sqrt(jnp.asarray(1.0 + EPS, _F32))
    h = _dott(h, we) + be
    z = g1 * (h * inv) + b1
    z = jnp.maximum(_dott(z, wm1) + bm1, 0.0)
    h = _dott(z, wm2) + bm2
    return g2 * (h * inv) + b2


def _xside_body(x_ref, we, be, g1, b1, wm1, bm1, wm2, bm2, g2, b2, w0a, b0,
                xe_ref, qn_ref, a0_ref):
    h = _embed_block(x_ref[...], we[...], be[...], g1[...], b1[...], wm1[...],
                     bm1[...], wm2[...], bm2[...], g2[...], b2[...])
    xe_ref[...] = h
    qn_ref[...] = jnp.sum(h * h, axis=1, keepdims=True)
    a0_ref[...] = _dott(h, w0a[...]) + b0[...]


def _xside(x, we, be, g1, b1, wm1, bm1, wm2, bm2, g2, b2, w0a, b0):
    return pl.pallas_call(
        _xside_body,
        out_shape=(
            jax.ShapeDtypeStruct((B, D), _F32),
            jax.ShapeDtypeStruct((B, 1), _F32),
            jax.ShapeDtypeStruct((B, 2 * D), _F32),
        ),
    )(x, we, be, g1, b1, wm1, bm1, wm2, bm2, g2, b2, w0a, b0)


def _cemb_body(cx_ref, we, be, g1, b1, wm1, bm1, wm2, bm2, g2, b2,
               ce_ref, cn_ref):
    h = _embed_block(cx_ref[...], we[...], be[...], g1[...], b1[...],
                     wm1[...], bm1[...], wm2[...], bm2[...], g2[...], b2[...])
    ce_ref[...] = h
    cn = jnp.sum(h * h, axis=1, keepdims=True)
    grow = (lax.broadcasted_iota(_I32, (1024, 1), 0)
            + pl.program_id(0) * 1024)
    cn_ref[...] = jnp.where(grow >= N, 1e30, cn)


def _cand_embed(cx_pad, we, be, g1, b1, wm1, bm1, wm2, bm2, g2, b2):
    nblk = NPAD // 1024
    w = lambda s: pl.BlockSpec(s, lambda i: (0,) * len(s))
    return pl.pallas_call(
        _cemb_body,
        grid=(nblk,),
        in_specs=[pl.BlockSpec((1024, D), lambda i: (i, 0)),
                  w((D, D)), w((1, D)), w((1, D)), w((1, D)), w((D, D)),
                  w((1, D)), w((D, D)), w((1, D)), w((1, D)), w((1, D))],
        out_specs=(pl.BlockSpec((1024, D), lambda i: (i, 0)),
                   pl.BlockSpec((1024, 1), lambda i: (i, 0))),
        out_shape=(jax.ShapeDtypeStruct((NPAD, D), _F32),
                   jax.ShapeDtypeStruct((NPAD, 1), _F32)),
        compiler_params=pltpu.CompilerParams(
            dimension_semantics=("parallel",)),
    )(cx_pad, we, be, g1, b1, wm1, bm1, wm2, bm2, g2, b2)


def _dist_body(xe_ref, qn_ref, ce_ref, cn_ref, d2_ref, gm_ref):
    x = xe_ref[...]
    c = ce_ref[...]
    s = _dott(x, c)
    d2 = (qn_ref[...] - 2.0 * s) + cn_ref[...]
    d2_ref[...] = d2
    gsm = jnp.concatenate(
        [jnp.min(d2[:, a * SG:(a + 1) * SG], axis=1, keepdims=True)
         for a in range(SPB)], axis=1)
    gm_ref[...] = jnp.concatenate(
        [gsm, jnp.full((QB, SG - SPB), jnp.inf, _F32)], axis=1)


def _dist(xe, qn, ce, cnr):
    return pl.pallas_call(
        _dist_body,
        grid=(B // QB, NPAD // CB),
        in_specs=[pl.BlockSpec((QB, D), lambda i, j: (i, 0)),
                  pl.BlockSpec((QB, 1), lambda i, j: (i, 0)),
                  pl.BlockSpec((CB, D), lambda i, j: (j, 0)),
                  pl.BlockSpec((1, CB), lambda i, j: (0, j))],
        out_specs=(pl.BlockSpec((QB, CB), lambda i, j: (i, j)),
                   pl.BlockSpec((QB, SG), lambda i, j: (i, j))),
        out_shape=(jax.ShapeDtypeStruct((B, NPAD), _F32),
                   jax.ShapeDtypeStruct((B, GMW), _F32)),
        compiler_params=pltpu.CompilerParams(
            dimension_semantics=("parallel", "parallel")),
    )(xe, qn, ce, cnr)


def _gtopk_body(gm_ref, gid_ref, pg_ref):
    v0 = gm_ref[...]
    lane = lax.broadcasted_iota(_I32, (QB, GMW), 1)
    col = lax.broadcasted_iota(_I32, (QB, K), 1)
    qrow = (lax.broadcasted_iota(_I32, (QB, 1), 0)
            + pl.program_id(0) * QB)
    big = jnp.int32(2 ** 30)
    zer = jnp.zeros((QB, K), _I32)

    def step(t, carry):
        v, gidm, pgm = carry
        m = jnp.min(v, axis=1, keepdims=True)
        am = jnp.min(jnp.where(v <= m, lane, big), axis=1, keepdims=True)
        sr = (am // SG) * SPB + am % SG
        sel = col == t
        gidm = jnp.where(sel, sr, gidm)
        pgm = jnp.where(sel, sr + qrow * NSG, pgm)
        v = jnp.where(lane == am, jnp.inf, v)
        return v, gidm, pgm

    _, gidm, pgm = lax.fori_loop(0, K, step, (v0, zer, zer), unroll=8)
    gid_ref[...] = gidm
    pg_ref[...] = pgm


def _gtopk(gm):
    return pl.pallas_call(
        _gtopk_body,
        grid=(B // QB,),
        in_specs=[pl.BlockSpec((QB, GMW), lambda i: (i, 0))],
        out_specs=(pl.BlockSpec((QB, K), lambda i: (i, 0)),
                   pl.BlockSpec((QB, K), lambda i: (i, 0))),
        out_shape=(jax.ShapeDtypeStruct((B, K), _I32),
                   jax.ShapeDtypeStruct((B, K), _I32)),
        compiler_params=pltpu.CompilerParams(
            dimension_semantics=("parallel",)),
    )(gm)


_NW = 32


def _sc_mesh():
    return plsc.VectorSubcoreMesh(core_axis_name="c", subcore_axis_name="s",
                                  num_cores=2, num_subcores=16)


def _sc_row_gather(tab, idx_flat, nrows_tab):
    nrows = B * K
    bpw = nrows // _NW
    npass = 4
    rpp = bpw // npass

    @functools.partial(
        pl.kernel, mesh=_sc_mesh(),
        out_type=jax.ShapeDtypeStruct((nrows, SG), _F32),
        scratch_types=[pltpu.VMEM((bpw,), _I32),
                       pltpu.VMEM((rpp, SG), _F32),
                       pltpu.SemaphoreType.DMA],
    )
    def k(tab_hbm, idx_hbm, out_hbm, idx_v, rows_v, sem):
        wid = lax.axis_index("s") * 2 + lax.axis_index("c")
        base = wid * bpw
        pltpu.sync_copy(idx_hbm.at[pl.ds(base, bpw)], idx_v)
        for p in range(npass):
            cps = [pltpu.async_copy(
                       tab_hbm.at[idx_v.at[pl.ds(p * rpp + ci * 128, 128)]],
                       rows_v.at[pl.ds(ci * 128, 128)], sem)
                   for ci in range(rpp // 128)]
            for cp in cps:
                cp.wait()
            pltpu.sync_copy(rows_v, out_hbm.at[pl.ds(base + p * rpp, rpp)])

    return k(tab, idx_flat)


def _pool_gather(d2rows, pgidx_flat):
    return _sc_row_gather(d2rows, pgidx_flat, B * NSG)


def _ptopk_body(pool_ref, gid_ref, idx_ref):
    v0 = pool_ref[...]
    g = gid_ref[...].astype(_F32)
    jj = lax.broadcasted_iota(_I32, (K, POOL), 0)
    ll = lax.broadcasted_iota(_I32, (K, POOL), 1)
    eye = jnp.where(ll // SG == jj, 1.0, 0.0).astype(_F32)
    gb = lax.dot_general(g, eye, (((1,), (0,)), ((), ())),
                         precision=lax.Precision.HIGHEST,
                         preferred_element_type=_F32)
    idxmat = ((gb + 0.5).astype(_I32) * SG
              + lax.broadcasted_iota(_I32, (QB, POOL), 1) % SG)
    big = jnp.int32(2 ** 30)
    col = lax.broadcasted_iota(_I32, (QB, K), 1)

    def step(t, carry):
        v, idxm = carry
        m = jnp.min(v, axis=1, keepdims=True)
        am = jnp.min(jnp.where(v <= m, idxmat, big), axis=1, keepdims=True)
        idxm = jnp.where(col == t, am, idxm)
        v = jnp.where(idxmat == am, jnp.inf, v)
        return v, idxm

    _, idxm = lax.fori_loop(0, K, step, (v0, jnp.zeros((QB, K), _I32)),
                            unroll=8)
    idx_ref[...] = idxm


def _ptopk(pool, gids):
    return pl.pallas_call(
        _ptopk_body,
        grid=(B // QB,),
        in_specs=[pl.BlockSpec((QB, POOL), lambda i: (i, 0)),
                  pl.BlockSpec((QB, K), lambda i: (i, 0))],
        out_specs=pl.BlockSpec((QB, K), lambda i: (i, 0)),
        out_shape=jax.ShapeDtypeStruct((B, K), _I32),
        compiler_params=pltpu.CompilerParams(
            dimension_semantics=("parallel",)),
    )(pool, gids)


def _ctx_gather(cand_emb, idx_flat):
    return _sc_row_gather(cand_emb, idx_flat, NPAD)


def _mlp_body(ctx_ref, a0_ref, w0b, w1, b1, wo, bo, out_ref):
    ctx = ctx_ref[...]
    a0 = a0_ref[...]
    rr = lax.broadcasted_iota(_I32, (ctx.shape[0], a0.shape[0]), 0) // K
    jj = lax.broadcasted_iota(_I32, (ctx.shape[0], a0.shape[0]), 1)
    rep = jnp.where(rr == jj, 1.0, 0.0).astype(_F32)
    aexp = lax.dot_general(rep, a0, (((1,), (0,)), ((), ())),
                           precision=lax.Precision.HIGHEST,
                           preferred_element_type=_F32)
    h1 = jnp.maximum(aexp + _dott(ctx, w0b[...]), 0.0)
    h2 = jnp.maximum(_dott(h1, w1[...]) + b1[...], 0.0)
    out_ref[...] = jnp.sum(h2 * wo[...], axis=1, keepdims=True) + bo[0]


def _mlp(ctx, a0, w0b, w1, b1, wo, bo):
    RB = 512
    w = lambda s: pl.BlockSpec(s, lambda i: (0,) * len(s))
    return pl.pallas_call(
        _mlp_body,
        grid=(B * K // RB,),
        in_specs=[pl.BlockSpec((RB, D), lambda i: (i, 0)),
                  pl.BlockSpec((RB // K, 2 * D), lambda i: (i, 0)),
                  w((2 * D, D)), w((2 * D, 2 * D)), w((1, 2 * D)),
                  w((1, 2 * D)),
                  pl.BlockSpec(memory_space=pltpu.SMEM)],
        out_specs=pl.BlockSpec((RB, 1), lambda i: (i, 0)),
        out_shape=jax.ShapeDtypeStruct((B * K, 1), _F32),
        compiler_params=pltpu.CompilerParams(
            dimension_semantics=("parallel",)),
    )(ctx, a0, w0b, w1, b1, wo, bo)


def kernel(x, candidate_x, W_embed, b_embed, bn1_g, bn1_b, W_mb1, b_mb1,
           W_mb2, b_mb2, bn2_g, bn2_b, W_mlp0, b_mlp0, W_mlp1, b_mlp1,
           W_out, b_out, is_train):
    del is_train
    row = lambda v: v.reshape(1, -1)
    be, g1, b1 = row(b_embed), row(bn1_g), row(bn1_b)
    bm1, bm2 = row(b_mb1), row(b_mb2)
    g2, b2 = row(bn2_g), row(bn2_b)
    w0a, w0b = W_mlp0[:, :D], W_mlp0[:, D:]
    b0, b1m, bo = row(b_mlp0), row(b_mlp1), b_out

    xe, qn, a0 = _xside(x, W_embed, be, g1, b1, W_mb1, bm1, W_mb2, bm2,
                        g2, b2, w0a, b0)
    cx_pad = jnp.pad(candidate_x, ((0, NPAD - N), (0, 0)))
    ce, cn = _cand_embed(cx_pad, W_embed, be, g1, b1, W_mb1, bm1, W_mb2, bm2,
                         g2, b2)
    d2, gm = _dist(xe, qn, ce, cn.reshape(1, NPAD))
    gids, pgidx = _gtopk(gm)
    pool = _pool_gather(d2.reshape(B * NSG, SG), pgidx.reshape(-1))
    idx = _ptopk(pool.reshape(B, POOL), gids)
    ctx = _ctx_gather(ce, idx.reshape(-1))
    out = _mlp(ctx, a0, w0b, W_mlp1, b1m, W_out, bo)
    return out.reshape(B, K, 1)


# speedup vs baseline: 7.5750x; 1.0934x over previous
import functools

import jax
import jax.numpy as jnp
from jax import lax
from jax.experimental import pallas as pl
from jax.experimental.pallas import tpu as pltpu
from jax.experimental.pallas import tpu_sc as plsc

B = 1024
N = 100000
D = 128
K = 32
EPS = 1e-5

NPAD = 102400
CB = 4096
QB = 256
SG = 128
NSG = NPAD // SG
SPB = CB // SG
GMW = (NPAD // CB) * SG
POOL = K * SG

_F32 = jnp.float32
_I32 = jnp.int32


def _dott(a, b):
    return lax.dot_general(a, b, (((1,), (1,)), ((), ())),
                           preferred_element_type=_F32)


def _embed_block(h, we, be, g1, b1, wm1, bm1, wm2, bm2, g2, b2):
    inv = 1.0 / jnp.sqrt(jnp.asarray(1.0 + EPS, _F32))
    h = _dott(h, we) + be
    z = g1 * (h * inv) + b1
    z = jnp.maximum(_dott(z, wm1) + bm1, 0.0)
    h = _dott(z, wm2) + bm2
    return g2 * (h * inv) + b2


def _xside_body(x_ref, we, be, g1, b1, wm1, bm1, wm2, bm2, g2, b2, w0a, b0,
                xe_ref, qn_ref, a0_ref):
    h = _embed_block(x_ref[...], we[...], be[...], g1[...], b1[...], wm1[...],
                     bm1[...], wm2[...], bm2[...], g2[...], b2[...])
    xe_ref[...] = h
    qn_ref[...] = jnp.sum(h * h, axis=1, keepdims=True)
    a0_ref[...] = _dott(h, w0a[...]) + b0[...]


def _xside(x, we, be, g1, b1, wm1, bm1, wm2, bm2, g2, b2, w0a, b0):
    return pl.pallas_call(
        _xside_body,
        out_shape=(
            jax.ShapeDtypeStruct((B, D), _F32),
            jax.ShapeDtypeStruct((B, 1), _F32),
            jax.ShapeDtypeStruct((B, 2 * D), _F32),
        ),
    )(x, we, be, g1, b1, wm1, bm1, wm2, bm2, g2, b2, w0a, b0)


def _cemb_body(cx_ref, we, be, g1, b1, wm1, bm1, wm2, bm2, g2, b2,
               ce_ref, cn_ref):
    h = _embed_block(cx_ref[...], we[...], be[...], g1[...], b1[...],
                     wm1[...], bm1[...], wm2[...], bm2[...], g2[...], b2[...])
    ce_ref[...] = h
    cn = jnp.sum(h * h, axis=1, keepdims=True)
    grow = (lax.broadcasted_iota(_I32, (1024, 1), 0)
            + pl.program_id(0) * 1024)
    cn_ref[...] = jnp.where(grow >= N, 1e30, cn)


def _cand_embed(cx_pad, we, be, g1, b1, wm1, bm1, wm2, bm2, g2, b2):
    nblk = NPAD // 1024
    w = lambda s: pl.BlockSpec(s, lambda i: (0,) * len(s))
    return pl.pallas_call(
        _cemb_body,
        grid=(nblk,),
        in_specs=[pl.BlockSpec((1024, D), lambda i: (i, 0)),
                  w((D, D)), w((1, D)), w((1, D)), w((1, D)), w((D, D)),
                  w((1, D)), w((D, D)), w((1, D)), w((1, D)), w((1, D))],
        out_specs=(pl.BlockSpec((1024, D), lambda i: (i, 0)),
                   pl.BlockSpec((1024, 1), lambda i: (i, 0))),
        out_shape=(jax.ShapeDtypeStruct((NPAD, D), _F32),
                   jax.ShapeDtypeStruct((NPAD, 1), _F32)),
        compiler_params=pltpu.CompilerParams(
            dimension_semantics=("parallel",)),
    )(cx_pad, we, be, g1, b1, wm1, bm1, wm2, bm2, g2, b2)


def _dist_body(xe_ref, qn_ref, ce_ref, cn_ref, d2_ref, gm_ref):
    x = xe_ref[...]
    c = ce_ref[...]
    s = _dott(x, c)
    d2 = (qn_ref[...] - 2.0 * s) + cn_ref[...]
    d2_ref[...] = d2
    gsm = jnp.concatenate(
        [jnp.min(d2[:, a * SG:(a + 1) * SG], axis=1, keepdims=True)
         for a in range(SPB)], axis=1)
    gm_ref[...] = gsm.T


def _dist(xe, qn, ce, cnr):
    return pl.pallas_call(
        _dist_body,
        grid=(B // QB, NPAD // CB),
        in_specs=[pl.BlockSpec((QB, D), lambda i, j: (i, 0)),
                  pl.BlockSpec((QB, 1), lambda i, j: (i, 0)),
                  pl.BlockSpec((CB, D), lambda i, j: (j, 0)),
                  pl.BlockSpec((1, CB), lambda i, j: (0, j))],
        out_specs=(pl.BlockSpec((QB, CB), lambda i, j: (i, j)),
                   pl.BlockSpec((SPB, QB), lambda i, j: (j, i))),
        out_shape=(jax.ShapeDtypeStruct((B, NPAD), _F32),
                   jax.ShapeDtypeStruct((NSG, B), _F32)),
        compiler_params=pltpu.CompilerParams(
            dimension_semantics=("parallel", "parallel")),
    )(xe, qn, ce, cnr)


def _gtopk_body(gm_ref, gid_ref, pg_ref):
    v0 = gm_ref[...]
    srow = lax.broadcasted_iota(_I32, (NSG, QB), 0)
    rowt = lax.broadcasted_iota(_I32, (K, QB), 0)
    qcol = (lax.broadcasted_iota(_I32, (1, QB), 1)
            + pl.program_id(0) * QB)
    big = jnp.int32(2 ** 30)
    zer = jnp.zeros((K, QB), _I32)

    def step(t, carry):
        v, gidm, pgm = carry
        m = jnp.min(v, axis=0, keepdims=True)
        am = jnp.min(jnp.where(v <= m, srow, big), axis=0, keepdims=True)
        sel = rowt == t
        gidm = jnp.where(sel, am, gidm)
        pgm = jnp.where(sel, am + qcol * NSG, pgm)
        v = jnp.where(srow == am, jnp.inf, v)
        return v, gidm, pgm

    _, gidm, pgm = lax.fori_loop(0, K, step, (v0, zer, zer), unroll=8)
    gid_ref[...] = gidm
    pg_ref[...] = pgm


def _gtopk(gm):
    return pl.pallas_call(
        _gtopk_body,
        grid=(B // QB,),
        in_specs=[pl.BlockSpec((NSG, QB), lambda i: (0, i))],
        out_specs=(pl.BlockSpec((K, QB), lambda i: (0, i)),
                   pl.BlockSpec((K, QB), lambda i: (0, i))),
        out_shape=(jax.ShapeDtypeStruct((K, B), _I32),
                   jax.ShapeDtypeStruct((K, B), _I32)),
        compiler_params=pltpu.CompilerParams(
            dimension_semantics=("parallel",)),
    )(gm)


_NW = 32


def _sc_mesh():
    return plsc.VectorSubcoreMesh(core_axis_name="c", subcore_axis_name="s",
                                  num_cores=2, num_subcores=16)


def _sc_row_gather(tab, idx_flat, nrows_tab):
    nrows = B * K
    bpw = nrows // _NW
    npass = 4
    rpp = bpw // npass

    @functools.partial(
        pl.kernel, mesh=_sc_mesh(),
        out_type=jax.ShapeDtypeStruct((nrows, SG), _F32),
        scratch_types=[pltpu.VMEM((bpw,), _I32),
                       pltpu.VMEM((rpp, SG), _F32),
                       pltpu.SemaphoreType.DMA],
    )
    def k(tab_hbm, idx_hbm, out_hbm, idx_v, rows_v, sem):
        wid = lax.axis_index("s") * 2 + lax.axis_index("c")
        base = wid * bpw
        pltpu.sync_copy(idx_hbm.at[pl.ds(base, bpw)], idx_v)
        for p in range(npass):
            cps = [pltpu.async_copy(
                       tab_hbm.at[idx_v.at[pl.ds(p * rpp + ci * 128, 128)]],
                       rows_v.at[pl.ds(ci * 128, 128)], sem)
                   for ci in range(rpp // 128)]
            for cp in cps:
                cp.wait()
            pltpu.sync_copy(rows_v, out_hbm.at[pl.ds(base + p * rpp, rpp)])

    return k(tab, idx_flat)


def _pool_gather(d2rows, pgidx_flat):
    return _sc_row_gather(d2rows, pgidx_flat, B * NSG)


def _ptopk_body(pool_ref, gid_ref, idx_ref):
    v0 = pool_ref[...]
    g = gid_ref[...].astype(_F32)
    jj = lax.broadcasted_iota(_I32, (K, POOL), 0)
    ll = lax.broadcasted_iota(_I32, (K, POOL), 1)
    eye = jnp.where(ll // SG == jj, 1.0, 0.0).astype(_F32)
    gb = lax.dot_general(g, eye, (((1,), (0,)), ((), ())),
                         precision=lax.Precision.HIGHEST,
                         preferred_element_type=_F32)
    idxmat = ((gb + 0.5).astype(_I32) * SG
              + lax.broadcasted_iota(_I32, (QB, POOL), 1) % SG)
    big = jnp.int32(2 ** 30)
    col = lax.broadcasted_iota(_I32, (QB, K), 1)

    def step(t, carry):
        v, idxm = carry
        m = jnp.min(v, axis=1, keepdims=True)
        am = jnp.min(jnp.where(v <= m, idxmat, big), axis=1, keepdims=True)
        idxm = jnp.where(col == t, am, idxm)
        v = jnp.where(idxmat == am, jnp.inf, v)
        return v, idxm

    _, idxm = lax.fori_loop(0, K, step, (v0, jnp.zeros((QB, K), _I32)),
                            unroll=8)
    idx_ref[...] = idxm


def _ptopk(pool, gids):
    return pl.pallas_call(
        _ptopk_body,
        grid=(B // QB,),
        in_specs=[pl.BlockSpec((QB, POOL), lambda i: (i, 0)),
                  pl.BlockSpec((QB, K), lambda i: (i, 0))],
        out_specs=pl.BlockSpec((QB, K), lambda i: (i, 0)),
        out_shape=jax.ShapeDtypeStruct((B, K), _I32),
        compiler_params=pltpu.CompilerParams(
            dimension_semantics=("parallel",)),
    )(pool, gids)


def _ctx_gather(cand_emb, idx_flat):
    return _sc_row_gather(cand_emb, idx_flat, NPAD)


def _mlp_body(ctx_ref, a0_ref, w0b, w1, b1, wo, bo, out_ref):
    ctx = ctx_ref[...]
    a0 = a0_ref[...]
    rr = lax.broadcasted_iota(_I32, (ctx.shape[0], a0.shape[0]), 0) // K
    jj = lax.broadcasted_iota(_I32, (ctx.shape[0], a0.shape[0]), 1)
    rep = jnp.where(rr == jj, 1.0, 0.0).astype(_F32)
    aexp = lax.dot_general(rep, a0, (((1,), (0,)), ((), ())),
                           precision=lax.Precision.HIGHEST,
                           preferred_element_type=_F32)
    h1 = jnp.maximum(aexp + _dott(ctx, w0b[...]), 0.0)
    h2 = jnp.maximum(_dott(h1, w1[...]) + b1[...], 0.0)
    out_ref[...] = jnp.sum(h2 * wo[...], axis=1, keepdims=True) + bo[0]


def _mlp(ctx, a0, w0b, w1, b1, wo, bo):
    RB = 512
    w = lambda s: pl.BlockSpec(s, lambda i: (0,) * len(s))
    return pl.pallas_call(
        _mlp_body,
        grid=(B * K // RB,),
        in_specs=[pl.BlockSpec((RB, D), lambda i: (i, 0)),
                  pl.BlockSpec((RB // K, 2 * D), lambda i: (i, 0)),
                  w((2 * D, D)), w((2 * D, 2 * D)), w((1, 2 * D)),
                  w((1, 2 * D)),
                  pl.BlockSpec(memory_space=pltpu.SMEM)],
        out_specs=pl.BlockSpec((RB, 1), lambda i: (i, 0)),
        out_shape=jax.ShapeDtypeStruct((B * K, 1), _F32),
        compiler_params=pltpu.CompilerParams(
            dimension_semantics=("parallel",)),
    )(ctx, a0, w0b, w1, b1, wo, bo)


def kernel(x, candidate_x, W_embed, b_embed, bn1_g, bn1_b, W_mb1, b_mb1,
           W_mb2, b_mb2, bn2_g, bn2_b, W_mlp0, b_mlp0, W_mlp1, b_mlp1,
           W_out, b_out, is_train):
    del is_train
    row = lambda v: v.reshape(1, -1)
    be, g1, b1 = row(b_embed), row(bn1_g), row(bn1_b)
    bm1, bm2 = row(b_mb1), row(b_mb2)
    g2, b2 = row(bn2_g), row(bn2_b)
    w0a, w0b = W_mlp0[:, :D], W_mlp0[:, D:]
    b0, b1m, bo = row(b_mlp0), row(b_mlp1), b_out

    xe, qn, a0 = _xside(x, W_embed, be, g1, b1, W_mb1, bm1, W_mb2, bm2,
                        g2, b2, w0a, b0)
    cx_pad = jnp.pad(candidate_x, ((0, NPAD - N), (0, 0)))
    ce, cn = _cand_embed(cx_pad, W_embed, be, g1, b1, W_mb1, bm1, W_mb2, bm2,
                         g2, b2)
    d2, gm = _dist(xe, qn, ce, cn.reshape(1, NPAD))
    gidt, pgt = _gtopk(gm)
    gids, pgidx = gidt.T, pgt.T
    pool = _pool_gather(d2.reshape(B * NSG, SG), pgidx.reshape(-1))
    idx = _ptopk(pool.reshape(B, POOL), gids)
    ctx = _ctx_gather(ce, idx.reshape(-1))
    out = _mlp(ctx, a0, w0b, W_mlp1, b1m, W_out, bo)
    return out.reshape(B, K, 1)
